# baseline (speedup 1.0000x reference)
import jax
import jax.numpy as jnp
from jax import lax
from jax.experimental import pallas as pl
from jax.experimental.pallas import tpu as pltpu
from jax.experimental.pallas import tpu_sc as plsc

NUM_E = 1000000
NUM_R = 1000
ED = 64
RD = 32
B = 16384
PF = ED * RD
PW = 2176

NC = 2
NS = 16
NW = NC * NS
BPW = B // NW
ECH = 128
PCH = 32

TB = 1024


def _tc_tr_body(src_ref, o_ref):
    o_ref[:, :ED] = src_ref[...].T
    o_ref[:, ED:] = jnp.zeros((TB, 128 - ED), jnp.float32)


def _tc_transpose(ent_t):
    g = (NUM_E + TB - 1) // TB
    return pl.pallas_call(
        _tc_tr_body,
        grid=(g,),
        in_specs=[pl.BlockSpec((ED, TB), lambda i: (0, i))],
        out_specs=pl.BlockSpec((TB, 128), lambda i: (i, 0)),
        out_shape=jax.ShapeDtypeStruct((NUM_E, 128), jnp.float32),
    )(ent_t)


def _sc_ent_body(heads_hbm, tails_hbm, ent_hbm, h_hbm, t_hbm,
                 idx_h, idx_t, gbuf, cbuf, sem):
    wid = lax.axis_index("s") * NC + lax.axis_index("c")
    base = wid * BPW
    pltpu.sync_copy(heads_hbm.at[pl.ds(base, BPW)], idx_h)
    pltpu.sync_copy(tails_hbm.at[pl.ds(base, BPW)], idx_t)

    @pl.loop(0, BPW, step=ECH)
    def _h(c):
        pltpu.async_copy(ent_hbm.at[idx_h.at[pl.ds(c, ECH)]], gbuf, sem).wait()
        pltpu.sync_copy(gbuf, h_hbm.at[pl.ds(base + c, ECH)])
        pltpu.async_copy(ent_hbm.at[idx_t.at[pl.ds(c, ECH)]], cbuf, sem).wait()
        pltpu.sync_copy(cbuf, t_hbm.at[pl.ds(base + c, ECH)])


def _sc_ent_gather(heads, tails, ent2):
    f32 = jnp.float32
    return pl.kernel(
        _sc_ent_body,
        out_type=(jax.ShapeDtypeStruct((B, 128), f32),
                  jax.ShapeDtypeStruct((B, 128), f32)),
        mesh=plsc.VectorSubcoreMesh(core_axis_name="c", subcore_axis_name="s"),
        scratch_types=[
            pltpu.VMEM((BPW,), jnp.int32),
            pltpu.VMEM((BPW,), jnp.int32),
            pltpu.VMEM((ECH, 128), f32),
            pltpu.VMEM((ECH, 128), f32),
            pltpu.SemaphoreType.DMA,
        ],
    )(heads, tails, ent2)


def _sc_proj_body(rels_hbm, proj_hbm, pg_hbm, idx_r, pbuf, sem):
    wid = lax.axis_index("s") * NC + lax.axis_index("c")
    base = wid * BPW
    pltpu.sync_copy(rels_hbm.at[pl.ds(base, BPW)], idx_r)

    @pl.loop(0, BPW, step=PCH)
    def _p(c):
        pltpu.async_copy(proj_hbm.at[idx_r.at[pl.ds(c, PCH)]], pbuf, sem).wait()
        pltpu.sync_copy(pbuf, pg_hbm.at[pl.ds(base + c, PCH)])


def _sc_proj_gather(relations, projaug):
    f32 = jnp.float32
    return pl.kernel(
        _sc_proj_body,
        out_type=jax.ShapeDtypeStruct((B, PW), f32),
        mesh=plsc.VectorSubcoreMesh(core_axis_name="c", subcore_axis_name="s"),
        scratch_types=[
            pltpu.VMEM((BPW,), jnp.int32),
            pltpu.VMEM((PCH, PW), f32),
            pltpu.SemaphoreType.DMA,
        ],
    )(relations, projaug)


TT = 256


def _tc_score_body(h_ref, t_ref, pg_ref, rsel_ref, o_ref):
    u = h_ref[:, :ED] - t_ref[:, :ED]
    ue = jax.lax.dot(u, rsel_ref[...],
                     preferred_element_type=jnp.float32)
    prod = ue * pg_ref[:, :PF]
    s = prod[:, 0:128]
    for c in range(1, PF // 128):
        s = s + prod[:, 128 * c:128 * (c + 1)]
    s4 = s[:, 0:32] + s[:, 32:64] + s[:, 64:96] + s[:, 96:128]
    diff = s4 + pg_ref[:, PF:PF + RD]
    o_ref[...] = jnp.sqrt(jnp.sum(diff * diff, axis=1))


def _tc_score(hg, tg, pg, rsel):
    return pl.pallas_call(
        _tc_score_body,
        grid=(B // TT,),
        in_specs=[
            pl.BlockSpec((TT, 128), lambda i: (i, 0)),
            pl.BlockSpec((TT, 128), lambda i: (i, 0)),
            pl.BlockSpec((TT, PW), lambda i: (i, 0)),
            pl.BlockSpec((ED, PF), lambda i: (0, 0)),
        ],
        out_specs=pl.BlockSpec((TT,), lambda i: (i,)),
        out_shape=jax.ShapeDtypeStruct((B,), jnp.float32),
    )(hg, tg, pg, rsel)


def kernel(heads, relations, tails, entity_embeddings, relation_embeddings,
           projection_matrices):
    projaug = jnp.concatenate(
        [projection_matrices.reshape(NUM_R, PF), relation_embeddings,
         jnp.zeros((NUM_R, PW - PF - RD), jnp.float32)], axis=1)
    k = jnp.arange(PF, dtype=jnp.int32)[None, :]
    d = jnp.arange(ED, dtype=jnp.int32)[:, None]
    rsel = (k // RD == d).astype(jnp.float32)
    ent2 = _tc_transpose(entity_embeddings.T)
    pg = _sc_proj_gather(relations, projaug)
    hg, tg = _sc_ent_gather(heads, tails, ent2)
    return _tc_score(hg, tg, pg, rsel)


# speedup vs baseline: 1.0149x; 1.0149x over previous
import jax
import jax.numpy as jnp
from jax import lax
from jax.experimental import pallas as pl
from jax.experimental.pallas import tpu as pltpu
from jax.experimental.pallas import tpu_sc as plsc

NUM_E = 1000000
NUM_R = 1000
ED = 64
RD = 32
B = 16384
PF = ED * RD
PW = 2176

NC = 2
NS = 16
NW = NC * NS
BPW = B // NW
ECH = 128
PCH = 32

TRB = 512
TRG = 977
SPLIT = TRG * TRB


def _tc_tr_body(srcl_ref, srcr_ref, o_ref):
    o_ref[:, :ED] = srcl_ref[...].T
    o_ref[:, ED:] = srcr_ref[...].T


def _tc_transpose(ent_t):
    return pl.pallas_call(
        _tc_tr_body,
        grid=(TRG,),
        in_specs=[pl.BlockSpec((ED, TRB), lambda i: (0, i)),
                  pl.BlockSpec((ED, TRB), lambda i: (0, i + TRG))],
        out_specs=pl.BlockSpec((TRB, 128), lambda i: (i, 0)),
        out_shape=jax.ShapeDtypeStruct((SPLIT, 128), jnp.float32),
    )(ent_t, ent_t)


def _sc_ent_body(heads_hbm, tails_hbm, ent_hbm, h_hbm, t_hbm,
                 idx_h, idx_t, gbuf, cbuf, sem):
    wid = lax.axis_index("s") * NC + lax.axis_index("c")
    base = wid * BPW
    pltpu.sync_copy(heads_hbm.at[pl.ds(base, BPW)], idx_h)
    pltpu.sync_copy(tails_hbm.at[pl.ds(base, BPW)], idx_t)

    @pl.loop(0, BPW, step=ECH)
    def _h(c):
        pltpu.async_copy(ent_hbm.at[idx_h.at[pl.ds(c, ECH)]], gbuf, sem).wait()
        pltpu.sync_copy(gbuf, h_hbm.at[pl.ds(base + c, ECH)])
        pltpu.async_copy(ent_hbm.at[idx_t.at[pl.ds(c, ECH)]], cbuf, sem).wait()
        pltpu.sync_copy(cbuf, t_hbm.at[pl.ds(base + c, ECH)])


def _sc_ent_gather(hrow, trow, ent2):
    f32 = jnp.float32
    return pl.kernel(
        _sc_ent_body,
        out_type=(jax.ShapeDtypeStruct((B, 128), f32),
                  jax.ShapeDtypeStruct((B, 128), f32)),
        mesh=plsc.VectorSubcoreMesh(core_axis_name="c", subcore_axis_name="s"),
        scratch_types=[
            pltpu.VMEM((BPW,), jnp.int32),
            pltpu.VMEM((BPW,), jnp.int32),
            pltpu.VMEM((ECH, 128), f32),
            pltpu.VMEM((ECH, 128), f32),
            pltpu.SemaphoreType.DMA,
        ],
    )(hrow, trow, ent2)


def _sc_proj_body(rels_hbm, proj_hbm, pg_hbm, idx_r, pbuf, sem):
    wid = lax.axis_index("s") * NC + lax.axis_index("c")
    base = wid * BPW
    pltpu.sync_copy(rels_hbm.at[pl.ds(base, BPW)], idx_r)

    @pl.loop(0, BPW, step=PCH)
    def _p(c):
        pltpu.async_copy(proj_hbm.at[idx_r.at[pl.ds(c, PCH)]], pbuf, sem).wait()
        pltpu.sync_copy(pbuf, pg_hbm.at[pl.ds(base + c, PCH)])


def _sc_proj_gather(relations, projaug):
    f32 = jnp.float32
    return pl.kernel(
        _sc_proj_body,
        out_type=jax.ShapeDtypeStruct((B, PW), f32),
        mesh=plsc.VectorSubcoreMesh(core_axis_name="c", subcore_axis_name="s"),
        scratch_types=[
            pltpu.VMEM((BPW,), jnp.int32),
            pltpu.VMEM((PCH, PW), f32),
            pltpu.SemaphoreType.DMA,
        ],
    )(relations, projaug)


TT = 256


def _tc_score_body(h_ref, t_ref, hs_ref, ts_ref, pg_ref, rsel_ref, o_ref):
    hrow = h_ref[...]
    trow = t_ref[...]
    hs = hs_ref[...]
    ts = ts_ref[...]
    h = hrow[:, :ED] + (hrow[:, ED:] - hrow[:, :ED]) * hs
    t = trow[:, :ED] + (trow[:, ED:] - trow[:, :ED]) * ts
    u = h - t
    ue = jax.lax.dot(u, rsel_ref[...],
                     preferred_element_type=jnp.float32)
    prod = ue * pg_ref[:, :PF]
    s = prod[:, 0:128]
    for c in range(1, PF // 128):
        s = s + prod[:, 128 * c:128 * (c + 1)]
    s4 = s[:, 0:32] + s[:, 32:64] + s[:, 64:96] + s[:, 96:128]
    diff = s4 + pg_ref[:, PF:PF + RD]
    o_ref[...] = jnp.sqrt(jnp.sum(diff * diff, axis=1))


def _tc_score(hg, tg, hs, ts, pg, rsel):
    return pl.pallas_call(
        _tc_score_body,
        grid=(B // TT,),
        in_specs=[
            pl.BlockSpec((TT, 128), lambda i: (i, 0)),
            pl.BlockSpec((TT, 128), lambda i: (i, 0)),
            pl.BlockSpec((TT, 1), lambda i: (i, 0)),
            pl.BlockSpec((TT, 1), lambda i: (i, 0)),
            pl.BlockSpec((TT, PW), lambda i: (i, 0)),
            pl.BlockSpec((ED, PF), lambda i: (0, 0)),
        ],
        out_specs=pl.BlockSpec((TT,), lambda i: (i,)),
        out_shape=jax.ShapeDtypeStruct((B,), jnp.float32),
    )(hg, tg, hs, ts, pg, rsel)


def kernel(heads, relations, tails, entity_embeddings, relation_embeddings,
           projection_matrices):
    projaug = jnp.concatenate(
        [projection_matrices.reshape(NUM_R, PF), relation_embeddings,
         jnp.zeros((NUM_R, PW - PF - RD), jnp.float32)], axis=1)
    k = jnp.arange(PF, dtype=jnp.int32)[None, :]
    d = jnp.arange(ED, dtype=jnp.int32)[:, None]
    rsel = (k // RD == d).astype(jnp.float32)
    ent2 = _tc_transpose(entity_embeddings.T)
    hrow = jnp.where(heads < SPLIT, heads, heads - SPLIT)
    trow = jnp.where(tails < SPLIT, tails, tails - SPLIT)
    hs = (heads >= SPLIT).astype(jnp.float32).reshape(B, 1)
    ts = (tails >= SPLIT).astype(jnp.float32).reshape(B, 1)
    pg = _sc_proj_gather(relations, projaug)
    hg, tg = _sc_ent_gather(hrow, trow, ent2)
    return _tc_score(hg, tg, hs, ts, pg, rsel)


# speedup vs baseline: 1.0709x; 1.0552x over previous
import jax
import jax.numpy as jnp
from jax import lax
from jax.experimental import pallas as pl
from jax.experimental.pallas import tpu as pltpu
from jax.experimental.pallas import tpu_sc as plsc

NUM_E = 1000000
NUM_R = 1000
ED = 64
RD = 32
B = 16384
PF = ED * RD
PW = 2176

NC = 2
NS = 16
NW = NC * NS
BPW = B // NW
ECH = 128
PCH = 32

TRB = 512
TRG = 977
SPLIT = TRG * TRB


def _tc_tr_body(srcl_ref, srcr_ref, o_ref):
    o_ref[:, :ED] = srcl_ref[...].T
    o_ref[:, ED:] = srcr_ref[...].T


def _tc_transpose(ent_t):
    return pl.pallas_call(
        _tc_tr_body,
        grid=(TRG,),
        in_specs=[pl.BlockSpec((ED, TRB), lambda i: (0, i)),
                  pl.BlockSpec((ED, TRB), lambda i: (0, i + TRG))],
        out_specs=pl.BlockSpec((TRB, 128), lambda i: (i, 0)),
        out_shape=jax.ShapeDtypeStruct((SPLIT, 128), jnp.float32),
    )(ent_t, ent_t)


def _sc_ent_body(heads_hbm, tails_hbm, ent_hbm, h_hbm, t_hbm,
                 idx_h, idx_t, gbuf, cbuf, sem):
    wid = lax.axis_index("s") * NC + lax.axis_index("c")
    base = wid * BPW
    pltpu.sync_copy(heads_hbm.at[pl.ds(base, BPW)], idx_h)
    pltpu.sync_copy(tails_hbm.at[pl.ds(base, BPW)], idx_t)

    @pl.loop(0, BPW, step=ECH)
    def _h(c):
        pltpu.async_copy(ent_hbm.at[idx_h.at[pl.ds(c, ECH)]], gbuf, sem).wait()
        pltpu.sync_copy(gbuf, h_hbm.at[pl.ds(base + c, ECH)])
        pltpu.async_copy(ent_hbm.at[idx_t.at[pl.ds(c, ECH)]], cbuf, sem).wait()
        pltpu.sync_copy(cbuf, t_hbm.at[pl.ds(base + c, ECH)])


def _sc_ent_gather(hrow, trow, ent2):
    f32 = jnp.float32
    return pl.kernel(
        _sc_ent_body,
        out_type=(jax.ShapeDtypeStruct((B, 128), f32),
                  jax.ShapeDtypeStruct((B, 128), f32)),
        mesh=plsc.VectorSubcoreMesh(core_axis_name="c", subcore_axis_name="s"),
        scratch_types=[
            pltpu.VMEM((BPW,), jnp.int32),
            pltpu.VMEM((BPW,), jnp.int32),
            pltpu.VMEM((ECH, 128), f32),
            pltpu.VMEM((ECH, 128), f32),
            pltpu.SemaphoreType.DMA,
        ],
    )(hrow, trow, ent2)


def _sc_proj_body(rels_hbm, proj_hbm, pg_hbm, idx_r, pbuf, sem):
    wid = lax.axis_index("s") * NC + lax.axis_index("c")
    base = wid * BPW
    pltpu.sync_copy(rels_hbm.at[pl.ds(base, BPW)], idx_r)

    @pl.loop(0, BPW, step=PCH)
    def _p(c):
        pltpu.async_copy(proj_hbm.at[idx_r.at[pl.ds(c, PCH)]], pbuf, sem).wait()
        pltpu.sync_copy(pbuf, pg_hbm.at[pl.ds(base + c, PCH)])


def _sc_proj_gather(relations, projaug):
    f32 = jnp.float32
    return pl.kernel(
        _sc_proj_body,
        out_type=jax.ShapeDtypeStruct((B, PW), f32),
        mesh=plsc.VectorSubcoreMesh(core_axis_name="c", subcore_axis_name="s"),
        scratch_types=[
            pltpu.VMEM((BPW,), jnp.int32),
            pltpu.VMEM((PCH, PW), f32),
            pltpu.SemaphoreType.DMA,
        ],
    )(relations, projaug)


TT = 256


def _tc_score_body(h_ref, t_ref, hs_ref, ts_ref, pg_ref, rsel_ref, o_ref):
    hrow = h_ref[...]
    trow = t_ref[...]
    hs = hs_ref[...]
    ts = ts_ref[...]
    h = hrow[:, :ED] + (hrow[:, ED:] - hrow[:, :ED]) * hs
    t = trow[:, :ED] + (trow[:, ED:] - trow[:, :ED]) * ts
    u = h - t
    ue = jax.lax.dot(u, rsel_ref[...],
                     preferred_element_type=jnp.float32)
    prod = ue * pg_ref[:, :PF]
    s = prod[:, 0:128]
    for c in range(1, PF // 128):
        s = s + prod[:, 128 * c:128 * (c + 1)]
    s4 = s[:, 0:32] + s[:, 32:64] + s[:, 64:96] + s[:, 96:128]
    diff = s4 + pg_ref[:, PF:PF + RD]
    o_ref[...] = jnp.sqrt(jnp.sum(diff * diff, axis=1))


def _tc_score(hg, tg, hs, ts, pg, rsel):
    return pl.pallas_call(
        _tc_score_body,
        grid=(B // TT,),
        in_specs=[
            pl.BlockSpec((TT, 128), lambda i: (i, 0)),
            pl.BlockSpec((TT, 128), lambda i: (i, 0)),
            pl.BlockSpec((TT, 1), lambda i: (i, 0)),
            pl.BlockSpec((TT, 1), lambda i: (i, 0)),
            pl.BlockSpec((TT, PW), lambda i: (i, 0)),
            pl.BlockSpec((ED, PF), lambda i: (0, 0)),
        ],
        out_specs=pl.BlockSpec((TT,), lambda i: (i,)),
        out_shape=jax.ShapeDtypeStruct((B,), jnp.float32),
    )(hg, tg, hs, ts, pg, rsel)


def kernel(heads, relations, tails, entity_embeddings, relation_embeddings,
           projection_matrices):
    projaug = jnp.concatenate(
        [projection_matrices.reshape(NUM_R, PF), relation_embeddings,
         jnp.zeros((NUM_R, PW - PF - RD), jnp.float32)], axis=1)
    k = jnp.arange(PF, dtype=jnp.int32)[None, :]
    d = jnp.arange(ED, dtype=jnp.int32)[:, None]
    rsel = (k // RD == d).astype(jnp.float32)
    ent2 = _tc_transpose(entity_embeddings.T)
    hrow = jnp.where(heads < SPLIT, heads, heads - SPLIT)
    trow = jnp.where(tails < SPLIT, tails, tails - SPLIT)
    hs = (heads >= SPLIT).astype(jnp.float32).reshape(B, 1)
    ts = (tails >= SPLIT).astype(jnp.float32).reshape(B, 1)
    pg = _sc_proj_gather(relations, projaug)
    hrow, trow, pg = lax.optimization_barrier((hrow, trow, pg))
    hg, tg = _sc_ent_gather(hrow, trow, ent2)
    return _tc_score(hg, tg, hs, ts, pg, rsel)
